# Initial kernel scaffold; baseline (speedup 1.0000x reference)
import jax
import jax.numpy as jnp
from jax.experimental import pallas as pl


def kernel(hidden_states, weight, e_score_correction_bias):
    raise NotImplementedError("write your pallas kernel here")


# speedup vs baseline: 2.3179x; 2.3179x over previous
import functools

import jax
import jax.numpy as jnp
from jax.experimental import pallas as pl

H = 7168
E = 256
TOPK = 8
N_GROUP = 8
TOPK_GROUP = 4
GROUP_SIZE = E // N_GROUP
SCALE = 2.5

BT = 256

NEG_INF = float("-inf")


def _gate_body(h_ref, wt_ref, b_ref, idx_ref, w_ref):
    logits = jnp.dot(h_ref[...], wt_ref[...],
                     preferred_element_type=jnp.float32)
    scores = jax.nn.sigmoid(logits)
    sc = scores + b_ref[...]

    gs_cols = []
    for g in range(N_GROUP):
        x = sc[:, g * GROUP_SIZE:(g + 1) * GROUP_SIZE]
        it = jax.lax.broadcasted_iota(jnp.int32, x.shape, 1)
        m1 = jnp.max(x, axis=1, keepdims=True)
        first = jnp.min(jnp.where(x == m1, it, GROUP_SIZE * 2),
                        axis=1, keepdims=True)
        m2 = jnp.max(jnp.where(it == first, NEG_INF, x),
                     axis=1, keepdims=True)
        gs_cols.append(m1 + m2)
    gs = jnp.concatenate(gs_cols, axis=1)

    git = jax.lax.broadcasted_iota(jnp.int32, gs.shape, 1)
    gmask = jnp.zeros(gs.shape, dtype=jnp.float32)
    work = gs
    for _ in range(TOPK_GROUP):
        m = jnp.max(work, axis=1, keepdims=True)
        sel = jnp.min(jnp.where(work == m, git, N_GROUP * 2),
                      axis=1, keepdims=True)
        hit = git == sel
        gmask = jnp.where(hit, 1.0, gmask)
        work = jnp.where(hit, NEG_INF, work)

    em = jnp.concatenate(
        [jnp.broadcast_to(gmask[:, g:g + 1], (gs.shape[0], GROUP_SIZE))
         for g in range(N_GROUP)], axis=1)
    masked = jnp.where(em > 0, sc, NEG_INF)

    eit = jax.lax.broadcasted_iota(jnp.int32, masked.shape, 1)
    idx_cols, w_cols = [], []
    work2 = masked
    for _ in range(TOPK):
        m = jnp.max(work2, axis=1, keepdims=True)
        sel = jnp.min(jnp.where(work2 == m, eit, E * 2),
                      axis=1, keepdims=True)
        hit = eit == sel
        wv = jnp.sum(jnp.where(hit, scores, 0.0), axis=1, keepdims=True)
        work2 = jnp.where(hit, NEG_INF, work2)
        idx_cols.append(sel)
        w_cols.append(wv)
    idxs = jnp.concatenate(idx_cols, axis=1)
    ws = jnp.concatenate(w_cols, axis=1)

    denom = jnp.sum(ws, axis=1, keepdims=True) + 1e-20
    w_ref[...] = ws / denom * SCALE
    idx_ref[...] = idxs


@jax.jit
def kernel(hidden_states, weight, e_score_correction_bias):
    t = hidden_states.shape[0]
    wt = weight.T
    bias = e_score_correction_bias.reshape(1, E)
    grid = (t // BT,)
    idx, w = pl.pallas_call(
        _gate_body,
        grid=grid,
        in_specs=[
            pl.BlockSpec((BT, H), lambda i: (i, 0)),
            pl.BlockSpec((H, E), lambda i: (0, 0)),
            pl.BlockSpec((1, E), lambda i: (0, 0)),
        ],
        out_specs=[
            pl.BlockSpec((BT, TOPK), lambda i: (i, 0)),
            pl.BlockSpec((BT, TOPK), lambda i: (i, 0)),
        ],
        out_shape=[
            jax.ShapeDtypeStruct((t, TOPK), jnp.int32),
            jax.ShapeDtypeStruct((t, TOPK), jnp.float32),
        ],
    )(hidden_states, wt, bias)
    return idx, w


# speedup vs baseline: 6.0337x; 2.6031x over previous
import jax
import jax.numpy as jnp
from jax.experimental import pallas as pl

H = 7168
E = 256
TOPK = 8
N_GROUP = 8
TOPK_GROUP = 4
GROUP_SIZE = E // N_GROUP
SCALE = 2.5

BT = 256

NEG_INF = float("-inf")


def _gate_body(h_ref, w_ref, b_ref, idx_ref, w_out_ref):
    logits = jax.lax.dot_general(
        w_ref[...], h_ref[...],
        (((1,), (1,)), ((), ())),
        preferred_element_type=jnp.float32)
    scores = jax.nn.sigmoid(logits)
    sc = scores + b_ref[...]

    bt = sc.shape[1]

    gs_rows = []
    for g in range(N_GROUP):
        x = sc[g * GROUP_SIZE:(g + 1) * GROUP_SIZE, :]
        it = jax.lax.broadcasted_iota(jnp.int32, x.shape, 0)
        m1 = jnp.max(x, axis=0, keepdims=True)
        first = jnp.min(jnp.where(x == m1, it, GROUP_SIZE * 2),
                        axis=0, keepdims=True)
        m2 = jnp.max(jnp.where(it == first, NEG_INF, x),
                     axis=0, keepdims=True)
        gs_rows.append(m1 + m2)
    gs = jnp.concatenate(gs_rows, axis=0)

    git = jax.lax.broadcasted_iota(jnp.int32, gs.shape, 0)
    gmask = jnp.zeros(gs.shape, dtype=jnp.float32)
    work = gs
    for _ in range(TOPK_GROUP):
        m = jnp.max(work, axis=0, keepdims=True)
        sel = jnp.min(jnp.where(work == m, git, N_GROUP * 2),
                      axis=0, keepdims=True)
        hit = git == sel
        gmask = jnp.where(hit, 1.0, gmask)
        work = jnp.where(hit, NEG_INF, work)

    em = jnp.concatenate(
        [jnp.broadcast_to(gmask[g:g + 1, :], (GROUP_SIZE, bt))
         for g in range(N_GROUP)], axis=0)
    masked = jnp.where(em > 0, sc, NEG_INF)

    eit = jax.lax.broadcasted_iota(jnp.int32, masked.shape, 0)
    idx_rows, w_rows = [], []
    work2 = masked
    for _ in range(TOPK):
        m = jnp.max(work2, axis=0, keepdims=True)
        sel = jnp.min(jnp.where(work2 == m, eit, E * 2),
                      axis=0, keepdims=True)
        hit = eit == sel
        wv = jnp.sum(jnp.where(hit, scores, 0.0), axis=0, keepdims=True)
        work2 = jnp.where(hit, NEG_INF, work2)
        idx_rows.append(sel)
        w_rows.append(wv)
    idxs = jnp.concatenate(idx_rows, axis=0)
    ws = jnp.concatenate(w_rows, axis=0)

    denom = jnp.sum(ws, axis=0, keepdims=True) + 1e-20
    w_out_ref[...] = ws / denom * SCALE
    idx_ref[...] = idxs


@jax.jit
def kernel(hidden_states, weight, e_score_correction_bias):
    t = hidden_states.shape[0]
    bias = e_score_correction_bias.reshape(E, 1)
    grid = (t // BT,)
    idx_t, w_t = pl.pallas_call(
        _gate_body,
        grid=grid,
        in_specs=[
            pl.BlockSpec((BT, H), lambda i: (i, 0)),
            pl.BlockSpec((E, H), lambda i: (0, 0)),
            pl.BlockSpec((E, 1), lambda i: (0, 0)),
        ],
        out_specs=[
            pl.BlockSpec((TOPK, BT), lambda i: (0, i)),
            pl.BlockSpec((TOPK, BT), lambda i: (0, i)),
        ],
        out_shape=[
            jax.ShapeDtypeStruct((TOPK, t), jnp.int32),
            jax.ShapeDtypeStruct((TOPK, t), jnp.float32),
        ],
    )(hidden_states, weight, bias)
    return idx_t.T, w_t.T


# speedup vs baseline: 6.7952x; 1.1262x over previous
import jax
import jax.numpy as jnp
from jax.experimental import pallas as pl

H = 7168
E = 256
TOPK = 8
N_GROUP = 8
TOPK_GROUP = 4
GROUP_SIZE = E // N_GROUP
SCALE = 2.5

BT = 512

NEG_INF = float("-inf")


def _gate_body(h_ref, w_ref, b_ref, idx_ref, w_out_ref):
    logits = jax.lax.dot_general(
        w_ref[...], h_ref[...],
        (((1,), (1,)), ((), ())),
        preferred_element_type=jnp.float32)
    scores = jax.nn.sigmoid(logits)
    sc = scores + b_ref[...]

    bt = sc.shape[1]

    gs_rows = []
    for g in range(N_GROUP):
        x = sc[g * GROUP_SIZE:(g + 1) * GROUP_SIZE, :]
        it = jax.lax.broadcasted_iota(jnp.int32, x.shape, 0)
        m1 = jnp.max(x, axis=0, keepdims=True)
        first = jnp.min(jnp.where(x == m1, it, GROUP_SIZE * 2),
                        axis=0, keepdims=True)
        m2 = jnp.max(jnp.where(it == first, NEG_INF, x),
                     axis=0, keepdims=True)
        gs_rows.append(m1 + m2)
    gs = jnp.concatenate(gs_rows, axis=0)

    git = jax.lax.broadcasted_iota(jnp.int32, gs.shape, 0)
    gmask = jnp.zeros(gs.shape, dtype=jnp.float32)
    work = gs
    for _ in range(TOPK_GROUP):
        m = jnp.max(work, axis=0, keepdims=True)
        sel = jnp.min(jnp.where(work == m, git, N_GROUP * 2),
                      axis=0, keepdims=True)
        hit = git == sel
        gmask = jnp.where(hit, 1.0, gmask)
        work = jnp.where(hit, NEG_INF, work)

    em = jnp.concatenate(
        [jnp.broadcast_to(gmask[g:g + 1, :], (GROUP_SIZE, bt))
         for g in range(N_GROUP)], axis=0)
    masked = jnp.where(em > 0, sc, NEG_INF)

    eit = jax.lax.broadcasted_iota(jnp.int32, masked.shape, 0)
    idx_rows, w_rows = [], []
    work2 = masked
    for _ in range(TOPK):
        m = jnp.max(work2, axis=0, keepdims=True)
        sel = jnp.min(jnp.where(work2 == m, eit, E * 2),
                      axis=0, keepdims=True)
        hit = eit == sel
        wv = jnp.sum(jnp.where(hit, scores, 0.0), axis=0, keepdims=True)
        work2 = jnp.where(hit, NEG_INF, work2)
        idx_rows.append(sel)
        w_rows.append(wv)
    idxs = jnp.concatenate(idx_rows, axis=0)
    ws = jnp.concatenate(w_rows, axis=0)

    denom = jnp.sum(ws, axis=0, keepdims=True) + 1e-20
    w_out_ref[...] = ws / denom * SCALE
    idx_ref[...] = idxs


@jax.jit
def kernel(hidden_states, weight, e_score_correction_bias):
    t = hidden_states.shape[0]
    bias = e_score_correction_bias.reshape(E, 1)
    grid = (t // BT,)
    idx_t, w_t = pl.pallas_call(
        _gate_body,
        grid=grid,
        in_specs=[
            pl.BlockSpec((BT, H), lambda i: (i, 0)),
            pl.BlockSpec((E, H), lambda i: (0, 0)),
            pl.BlockSpec((E, 1), lambda i: (0, 0)),
        ],
        out_specs=[
            pl.BlockSpec((TOPK, BT), lambda i: (0, i)),
            pl.BlockSpec((TOPK, BT), lambda i: (0, i)),
        ],
        out_shape=[
            jax.ShapeDtypeStruct((TOPK, t), jnp.int32),
            jax.ShapeDtypeStruct((TOPK, t), jnp.float32),
        ],
    )(hidden_states, weight, bias)
    return idx_t.T, w_t.T
